# baseline (speedup 1.0000x reference)
import jax
import jax.numpy as jnp
from jax import lax
from jax.experimental import pallas as pl
from jax.experimental.pallas import tpu as pltpu
from jax.experimental.pallas import tpu_sc as plsc

SEQ_LEN = 8192
EMB = 1024
BSZ = 4

NUM_CORES = 2
NUM_SUBCORES = 16
NUM_WORKERS = NUM_CORES * NUM_SUBCORES
ROWS_PER_WORKER = SEQ_LEN // NUM_WORKERS
LANES = 16
CHUNK = 64
NUM_CHUNKS = ROWS_PER_WORKER // CHUNK
VECS = CHUNK // LANES


def _sc_body(len_hbm, table_hbm, out_hbm, len_v, rows_v, gsem):
    wid = lax.axis_index("s") * NUM_CORES + lax.axis_index("c")
    base = wid * ROWS_PER_WORKER
    pltpu.sync_copy(len_hbm, len_v)
    start = len_v[...]
    lane = lax.iota(jnp.int32, LANES)
    for c in range(NUM_CHUNKS):
        off = base + c * CHUNK
        gathers = []
        for j in range(VECS):
            vec = jnp.clip(start + (lane + (off + j * LANES)),
                           0, SEQ_LEN - 1)
            gathers.append(pltpu.async_copy(
                table_hbm.at[vec],
                rows_v.at[pl.ds(j * LANES, LANES)], gsem))
        for g in gathers:
            g.wait()
        for b in range(BSZ):
            pltpu.sync_copy(rows_v, out_hbm.at[b, pl.ds(off, CHUNK)])


def kernel(inputs, embedding_table, length=0):
    del inputs
    len_vec = jnp.full((LANES,), jnp.asarray(length, jnp.int32))
    mesh = plsc.VectorSubcoreMesh(
        core_axis_name="c", subcore_axis_name="s")
    run = pl.kernel(
        _sc_body,
        out_type=jax.ShapeDtypeStruct((BSZ, SEQ_LEN, EMB), jnp.float32),
        mesh=mesh,
        scratch_types=[
            pltpu.VMEM((LANES,), jnp.int32),
            pltpu.VMEM((CHUNK, EMB), jnp.float32),
            pltpu.SemaphoreType.DMA,
        ],
    )
    return run(len_vec, embedding_table)


# speedup vs baseline: 1.0471x; 1.0471x over previous
import jax
import jax.numpy as jnp
from jax import lax
from jax.experimental import pallas as pl
from jax.experimental.pallas import tpu as pltpu
from jax.experimental.pallas import tpu_sc as plsc

SEQ_LEN = 8192
EMB = 1024
BSZ = 4

NUM_CORES = 2
NUM_SUBCORES = 16
NUM_WORKERS = NUM_CORES * NUM_SUBCORES
ROWS_PER_WORKER = SEQ_LEN // NUM_WORKERS
CHUNK = 64
NUM_CHUNKS = ROWS_PER_WORKER // CHUNK


def _sc_body(idx_hbm, table_hbm, out_hbm, idx_v, rows_v, gsem):
    wid = lax.axis_index("s") * NUM_CORES + lax.axis_index("c")
    base = wid * ROWS_PER_WORKER
    pltpu.sync_copy(idx_hbm.at[pl.ds(base, ROWS_PER_WORKER)], idx_v)
    for c in range(NUM_CHUNKS):
        off = base + c * CHUNK
        pltpu.async_copy(
            table_hbm.at[idx_v.at[pl.ds(c * CHUNK, CHUNK)]],
            rows_v, gsem).wait()
        for b in range(BSZ):
            pltpu.sync_copy(rows_v, out_hbm.at[b, pl.ds(off, CHUNK)])


def kernel(inputs, embedding_table, length=0):
    del inputs
    seq = jnp.arange(SEQ_LEN, dtype=jnp.int32) + jnp.asarray(
        length, dtype=jnp.int32)
    idx = jnp.clip(seq, 0, SEQ_LEN - 1)
    mesh = plsc.VectorSubcoreMesh(
        core_axis_name="c", subcore_axis_name="s")
    run = pl.kernel(
        _sc_body,
        out_type=jax.ShapeDtypeStruct((BSZ, SEQ_LEN, EMB), jnp.float32),
        mesh=mesh,
        scratch_types=[
            pltpu.VMEM((ROWS_PER_WORKER,), jnp.int32),
            pltpu.VMEM((CHUNK, EMB), jnp.float32),
            pltpu.SemaphoreType.DMA,
        ],
    )
    return run(idx, embedding_table)
